# baseline (speedup 1.0000x reference)
import functools
import math

import jax
import jax.numpy as jnp
from jax import lax
from jax.experimental import pallas as pl
from jax.experimental.pallas import tpu as pltpu
from jax.experimental.pallas import tpu_sc as plsc

BATCH = 4096
HIST = 200
D = 64
NC, NS = 2, 16
NW = NC * NS
BW = BATCH // NW
HG = 8
NHG = HIST // HG
SCALE = math.sqrt(D)

_mesh = plsc.VectorSubcoreMesh(
    core_axis_name="c", subcore_axis_name="s", num_cores=NC, num_subcores=NS
)


@functools.partial(
    pl.kernel,
    mesh=_mesh,
    out_type=jax.ShapeDtypeStruct((HIST, D, BATCH), jnp.float32),
    scratch_types=[
        pltpu.VMEM((HG, 128), jnp.int32),
        pltpu.VMEM((HG, 128), jnp.int32),
        pltpu.VMEM((2, 128, 128), jnp.float32),
        pltpu.VMEM((2, D, 128), jnp.float32),
        pltpu.SemaphoreType.DMA,
        pltpu.SemaphoreType.DMA,
        pltpu.SemaphoreType.DMA,
        pltpu.SemaphoreType.DMA,
    ],
    compiler_params=pltpu.CompilerParams(needs_layout_passes=False),
)
def _embed_sc(table2_hbm, srcT_hbm, out_hbm, idx_v, idx2_v, rows_v, outT_v,
              gsem0, gsem1, osem0, osem1):
    wid = lax.axis_index("s") * NC + lax.axis_index("c")
    b0 = wid * BW
    gsems = [gsem0, gsem1]
    osems = [osem0, osem1]
    lane = lax.iota(jnp.int32, 16)
    rowsel = [lane + jb * 16 for jb in range(8)]

    def hg_body(hg, carry):
        h_base = hg * HG
        pltpu.sync_copy(srcT_hbm.at[pl.ds(h_base, HG), pl.ds(b0, BW)], idx_v)

        def shift_body(k, c2):
            r = k // 8
            sl = pl.ds((k % 8) * 16, 16)
            idx2_v[r, sl] = idx_v[r, sl] >> 1
            return c2

        lax.fori_loop(0, HG * 8, shift_body, 0, unroll=4)

        def fire_gather(h8):
            return pltpu.async_copy(
                table2_hbm.at[idx2_v.at[h8]], rows_v.at[h8 % 2],
                gsems[h8 % 2],
            )

        gh = [None] * HG
        oh = [None] * HG
        gh[0] = fire_gather(0)
        for h8 in range(HG):
            cur = h8 % 2
            if h8 + 1 < HG:
                gh[h8 + 1] = fire_gather(h8 + 1)
            gh[h8].wait()
            if h8 >= 2:
                oh[h8 - 2].wait()
            hv = [(idx_v[h8, pl.ds(jb * 16, 16)] & 1) * D for jb in range(8)]

            def d_body(d, c2):
                for jb in range(8):
                    vals = plsc.load_gather(
                        rows_v.at[cur], [rowsel[jb], hv[jb] + d]
                    )
                    outT_v[cur, d, pl.ds(jb * 16, 16)] = vals * SCALE
                return c2

            lax.fori_loop(0, D, d_body, 0, unroll=2)
            oh[h8] = pltpu.async_copy(
                outT_v.at[cur],
                out_hbm.at[h_base + h8, :, pl.ds(b0, BW)],
                osems[cur],
            )
        oh[HG - 2].wait()
        oh[HG - 1].wait()
        return carry

    lax.fori_loop(0, NHG, hg_body, 0)


def kernel(source, table):
    srcT = source.astype(jnp.int32).T
    table2 = table.reshape(500000, 128)
    out3 = _embed_sc(table2, srcT)
    return out3.transpose(2, 0, 1)


# speedup vs baseline: 1.6374x; 1.6374x over previous
import functools
import math

import jax
import jax.numpy as jnp
from jax import lax
from jax.experimental import pallas as pl
from jax.experimental.pallas import tpu as pltpu
from jax.experimental.pallas import tpu_sc as plsc

BATCH = 4096
HIST = 200
D = 64
B = BATCH * HIST
NC, NS = 2, 16
NW = NC * NS
ROWS_PER_W = B // NW
CHUNK = 512
IPC = CHUNK // 128
NCHUNK = ROWS_PER_W // CHUNK
NBUF = 3
SCALE = math.sqrt(D)

_mesh = plsc.VectorSubcoreMesh(
    core_axis_name="c", subcore_axis_name="s", num_cores=NC, num_subcores=NS
)


@functools.partial(
    pl.kernel,
    mesh=_mesh,
    out_type=jax.ShapeDtypeStruct((B, D), jnp.float32),
    scratch_types=[
        pltpu.VMEM((NBUF, IPC, 128), jnp.int32),
        pltpu.VMEM((NBUF, CHUNK, D), jnp.float32),
        pltpu.SemaphoreType.DMA,
        pltpu.SemaphoreType.DMA,
        pltpu.SemaphoreType.DMA,
        pltpu.SemaphoreType.DMA,
        pltpu.SemaphoreType.DMA,
        pltpu.SemaphoreType.DMA,
    ],
    compiler_params=pltpu.CompilerParams(use_tc_tiling_on_sc=False),
)
def _embed_sc(table_hbm, src_hbm, out_hbm, idx_v, rows_v,
              g0, g1, g2, o0, o1, o2):
    wid = lax.axis_index("s") * NC + lax.axis_index("c")
    chunk0 = wid * NCHUNK
    base = wid * ROWS_PER_W
    gsems = [g0, g1, g2]
    osems = [o0, o1, o2]

    def load_idx(i):
        pltpu.sync_copy(src_hbm.at[chunk0 + i], idx_v.at[i % NBUF])

    def fire_gather(i):
        b = i % NBUF
        return [
            pltpu.async_copy(
                table_hbm.at[idx_v.at[b, j]],
                rows_v.at[b, pl.ds(j * 128, 128)],
                gsems[b],
            )
            for j in range(IPC)
        ]

    gh = [None] * NCHUNK
    oh = [None] * NCHUNK
    load_idx(0)
    gh[0] = fire_gather(0)
    load_idx(1)
    gh[1] = fire_gather(1)

    for i in range(NCHUNK):
        b = i % NBUF
        for h in gh[i]:
            h.wait()

        def scale_row(r, c2):
            for j in range(D // 16):
                sl = pl.ds(j * 16, 16)
                rows_v[b, r, sl] = rows_v[b, r, sl] * SCALE
            return c2

        lax.fori_loop(0, CHUNK, scale_row, 0, unroll=4)
        oh[i] = pltpu.async_copy(
            rows_v.at[b], out_hbm.at[pl.ds(base + i * CHUNK, CHUNK)], osems[b]
        )
        if i + 2 < NCHUNK:
            load_idx(i + 2)
            if i >= 1:
                oh[i - 1].wait()
            gh[i + 2] = fire_gather(i + 2)
    oh[NCHUNK - 3].wait()
    oh[NCHUNK - 2].wait()
    oh[NCHUNK - 1].wait()


def kernel(source, table):
    src = source.astype(jnp.int32).reshape(B // CHUNK, IPC, 128)
    out = _embed_sc(table, src)
    return out.reshape(BATCH, HIST, D)
